# Initial kernel scaffold; baseline (speedup 1.0000x reference)
import jax
import jax.numpy as jnp
from jax.experimental import pallas as pl


def kernel(x, edge_index, W1, b1, gamma, beta, running_mean, running_var, W2, b2):
    raise NotImplementedError("write your pallas kernel here")


# speedup vs baseline: 16.1081x; 16.1081x over previous
import functools

import jax
import jax.numpy as jnp
from jax import lax
from jax.experimental import pallas as pl
from jax.experimental.pallas import tpu as pltpu
from jax.experimental.pallas import tpu_sc as plsc

N = 10000
E = 320000
D_IN = 128
D_HID = 32
D_OUT = 128

CW = 80
NSC = 2
NTEC = 16
CPT = E // (NSC * NTEC * CW)
WB = 10
RZ = N // WB

_mesh = plsc.VectorSubcoreMesh(core_axis_name="c", subcore_axis_name="s")


@functools.partial(
    pl.kernel,
    out_type=jax.ShapeDtypeStruct((NSC * N,), jnp.float32),
    mesh=_mesh,
    scratch_types=[
        pltpu.VMEM((CW,), jnp.int32),
        pltpu.VMEM((CW,), jnp.float32),
        pltpu.VMEM((RZ,), jnp.float32),
        pltpu.VMEM_SHARED((N,), jnp.float32),
        pltpu.SemaphoreType.DMA,
    ],
)
def _sc_degree(dst_hbm, zeros_hbm, deg_hbm, dst_v, ones_v, stage_v, acc, sem):
    c = lax.axis_index("c")
    s = lax.axis_index("s")
    w = c * NTEC + s

    @pl.when(s < WB)
    def _():
        pltpu.sync_copy(zeros_hbm.at[pl.ds(s * RZ, RZ)], stage_v)
        pltpu.sync_copy(stage_v, acc.at[pl.ds(s * RZ, RZ)])

    for i in range(CW // 16):
        ones_v[pl.ds(i * 16, 16)] = jnp.full((16,), 1.0, jnp.float32)
    plsc.subcore_barrier()

    def body(j, carry):
        base = (w * CPT + j) * CW
        pltpu.sync_copy(dst_hbm.at[pl.ds(base, CW)], dst_v)
        pltpu.sync_copy(ones_v, acc.at[dst_v], add=True)
        return carry

    lax.fori_loop(0, CPT, body, 0)
    plsc.subcore_barrier()

    @pl.when(s < WB)
    def _():
        pltpu.sync_copy(acc.at[pl.ds(s * RZ, RZ)], stage_v)
        pltpu.sync_copy(stage_v, deg_hbm.at[pl.ds(c * N + s * RZ, RZ)])


@functools.partial(
    pl.kernel,
    out_type=jax.ShapeDtypeStruct((NSC, WB, RZ, D_HID), jnp.float32),
    mesh=_mesh,
    scratch_types=[
        pltpu.VMEM((CW,), jnp.int32),
        pltpu.VMEM((CW,), jnp.int32),
        pltpu.VMEM((CW, D_HID), jnp.float32),
        pltpu.VMEM((RZ, D_HID), jnp.float32),
        pltpu.VMEM_SHARED((N, D_HID), jnp.float32),
        pltpu.SemaphoreType.DMA,
    ],
    compiler_params=pltpu.CompilerParams(use_tc_tiling_on_sc=False),
)
def _sc_edge_agg(g_hbm, src_hbm, dst_hbm, zeros_hbm, out_hbm,
                 src_v, dst_v, rows_v, stage_v, acc, sem):
    c = lax.axis_index("c")
    s = lax.axis_index("s")
    w = c * NTEC + s

    @pl.when(s < WB)
    def _():
        pltpu.sync_copy(zeros_hbm.at[pl.ds(s * RZ, RZ)], stage_v)
        pltpu.sync_copy(stage_v, acc.at[pl.ds(s * RZ, RZ)])

    plsc.subcore_barrier()

    def body(j, carry):
        base = (w * CPT + j) * CW
        pltpu.sync_copy(src_hbm.at[pl.ds(base, CW)], src_v)
        pltpu.sync_copy(dst_hbm.at[pl.ds(base, CW)], dst_v)
        pltpu.async_copy(g_hbm.at[src_v], rows_v, sem).wait()
        pltpu.sync_copy(rows_v, acc.at[dst_v], add=True)
        return carry

    lax.fori_loop(0, CPT, body, 0)
    plsc.subcore_barrier()

    @pl.when(s < WB)
    def _():
        pltpu.sync_copy(acc.at[pl.ds(s * RZ, RZ)], stage_v)
        pltpu.sync_copy(stage_v, out_hbm.at[c, s])


def _tc_prep_body(degp_ref, x_ref, w1_ref, dinv_ref, g1_ref):
    deg = degp_ref[0] + degp_ref[1] + 1.0
    dinv = lax.rsqrt(deg)
    dinv_ref[...] = dinv
    g1_ref[...] = jnp.dot(x_ref[...] * dinv, w1_ref[...],
                          preferred_element_type=jnp.float32)


_tc_prep = pl.pallas_call(
    _tc_prep_body,
    out_shape=(jax.ShapeDtypeStruct((N, 1), jnp.float32),
               jax.ShapeDtypeStruct((N, D_HID), jnp.float32)),
)


def _tc_mid_body(accp_ref, g1_ref, dinv_ref, b1_ref, gamma_ref, beta_ref,
                 mean_ref, var_ref, s_ref):
    dinv = dinv_ref[...]
    pre = dinv * (accp_ref[0] + accp_ref[1] + g1_ref[...]) + b1_ref[...]
    scale = gamma_ref[...] * lax.rsqrt(var_ref[...] + 1e-5)
    hb = (pre - mean_ref[...]) * scale + beta_ref[...]
    s_ref[...] = dinv * jnp.maximum(hb, 0.0)


_tc_mid = pl.pallas_call(
    _tc_mid_body,
    out_shape=jax.ShapeDtypeStruct((N, D_HID), jnp.float32),
)


def _tc_out_body(accp_ref, s_ref, dinv_ref, w2_ref, b2_ref, out_ref):
    u = dinv_ref[...] * (accp_ref[0] + accp_ref[1] + s_ref[...])
    out_ref[...] = jnp.dot(u, w2_ref[...],
                           preferred_element_type=jnp.float32) + b2_ref[...]


_tc_out = pl.pallas_call(
    _tc_out_body,
    out_shape=jax.ShapeDtypeStruct((N, D_OUT), jnp.float32),
)


@jax.jit
def kernel(x, edge_index, W1, b1, gamma, beta, running_mean, running_var,
           W2, b2):
    src = edge_index[0]
    dst = edge_index[1]
    z_n = jnp.zeros((N,), jnp.float32)
    z_h = jnp.zeros((N, D_HID), jnp.float32)

    degp = _sc_degree(dst, z_n).reshape(NSC, N, 1)
    dinv, g1 = _tc_prep(degp, x, W1)
    acc1 = _sc_edge_agg(g1, src, dst, z_h).reshape(NSC, N, D_HID)
    s = _tc_mid(acc1, g1, dinv, b1.reshape(1, -1), gamma.reshape(1, -1),
                beta.reshape(1, -1), running_mean.reshape(1, -1),
                running_var.reshape(1, -1))
    acc2 = _sc_edge_agg(s, src, dst, z_h).reshape(NSC, N, D_HID)
    return _tc_out(acc2, s, dinv, W2, b2.reshape(1, -1))


# speedup vs baseline: 54.4365x; 3.3795x over previous
import functools

import jax
import jax.numpy as jnp
from jax import lax
from jax.experimental import pallas as pl
from jax.experimental.pallas import tpu as pltpu
from jax.experimental.pallas import tpu_sc as plsc

N = 10000
E = 320000
D_IN = 128
D_HID = 32
D_OUT = 128

CW = 80
NSC = 2
NTEC = 16
CPT = E // (NSC * NTEC * CW)
NBUF = 5
WB = 10
RZ = N // WB

_mesh = plsc.VectorSubcoreMesh(core_axis_name="c", subcore_axis_name="s")


@functools.partial(
    pl.kernel,
    out_type=jax.ShapeDtypeStruct((NSC * N,), jnp.float32),
    mesh=_mesh,
    scratch_types=[
        pltpu.VMEM((CPT, CW), jnp.int32),
        pltpu.VMEM((CW,), jnp.float32),
        pltpu.VMEM((RZ,), jnp.float32),
        pltpu.VMEM_SHARED((N,), jnp.float32),
        pltpu.SemaphoreType.DMA,
    ],
    compiler_params=pltpu.CompilerParams(use_tc_tiling_on_sc=False),
)
def _sc_degree(dst_hbm, zeros_hbm, deg_hbm, dst_v, ones_v, stage_v, acc, sem):
    c = lax.axis_index("c")
    s = lax.axis_index("s")
    w = c * NTEC + s

    @pl.when(s < WB)
    def _():
        pltpu.sync_copy(zeros_hbm.at[pl.ds(s * RZ, RZ)], stage_v)
        pltpu.sync_copy(stage_v, acc.at[pl.ds(s * RZ, RZ)])

    for i in range(CW // 16):
        ones_v[pl.ds(i * 16, 16)] = jnp.full((16,), 1.0, jnp.float32)
    pltpu.sync_copy(dst_hbm.at[pl.ds(w * CPT, CPT)], dst_v)
    plsc.subcore_barrier()

    def fire(j, carry):
        pltpu.async_copy(ones_v, acc.at[dst_v.at[j]], sem, add=True)
        return carry

    lax.fori_loop(0, CPT, fire, 0)

    def drain(j, carry):
        pltpu.make_async_copy(ones_v, acc.at[dst_v.at[0]], sem).wait()
        return carry

    lax.fori_loop(0, CPT, drain, 0)
    plsc.subcore_barrier()

    @pl.when(s < WB)
    def _():
        pltpu.sync_copy(acc.at[pl.ds(s * RZ, RZ)], stage_v)
        pltpu.sync_copy(stage_v, deg_hbm.at[pl.ds(c * N + s * RZ, RZ)])


@functools.partial(
    pl.kernel,
    out_type=jax.ShapeDtypeStruct((NSC, WB, RZ, D_HID), jnp.float32),
    mesh=_mesh,
    scratch_types=[
        pltpu.VMEM((CPT, CW), jnp.int32),
        pltpu.VMEM((CPT, CW), jnp.int32),
        pltpu.VMEM((NBUF, CW, D_HID), jnp.float32),
        pltpu.VMEM((RZ, D_HID), jnp.float32),
        pltpu.VMEM_SHARED((N, D_HID), jnp.float32),
    ] + [pltpu.SemaphoreType.DMA] * NBUF,
    compiler_params=pltpu.CompilerParams(use_tc_tiling_on_sc=False),
)
def _sc_edge_agg(g_hbm, src_hbm, dst_hbm, zeros_hbm, out_hbm,
                 src_v, dst_v, rows_v, stage_v, acc, *sems):
    c = lax.axis_index("c")
    s = lax.axis_index("s")
    w = c * NTEC + s

    @pl.when(s < WB)
    def _():
        pltpu.sync_copy(zeros_hbm.at[pl.ds(s * RZ, RZ)], stage_v)
        pltpu.sync_copy(stage_v, acc.at[pl.ds(s * RZ, RZ)])

    pltpu.sync_copy(src_hbm.at[pl.ds(w * CPT, CPT)], src_v)
    pltpu.sync_copy(dst_hbm.at[pl.ds(w * CPT, CPT)], dst_v)
    plsc.subcore_barrier()

    for b in range(NBUF):
        pltpu.async_copy(g_hbm.at[src_v.at[b]], rows_v.at[b], sems[b])

    def outer(i, carry):
        for b in range(NBUF):
            j = i * NBUF + b
            pltpu.make_async_copy(g_hbm.at[src_v.at[j]], rows_v.at[b],
                                  sems[b]).wait()
            pltpu.sync_copy(rows_v.at[b], acc.at[dst_v.at[j]], add=True)

            @pl.when(i < CPT // NBUF - 1)
            def _():
                pltpu.async_copy(g_hbm.at[src_v.at[j + NBUF]], rows_v.at[b],
                                 sems[b])
        return carry

    lax.fori_loop(0, CPT // NBUF, outer, 0)
    plsc.subcore_barrier()

    @pl.when(s < WB)
    def _():
        pltpu.sync_copy(acc.at[pl.ds(s * RZ, RZ)], stage_v)
        pltpu.sync_copy(stage_v, out_hbm.at[c, s])


def _tc_prep_body(degp_ref, x_ref, w1_ref, dinv_ref, g1_ref):
    deg = degp_ref[0] + degp_ref[1] + 1.0
    dinv = lax.rsqrt(deg)
    dinv_ref[...] = dinv
    g1_ref[...] = jnp.dot(x_ref[...] * dinv, w1_ref[...],
                          preferred_element_type=jnp.float32)


_tc_prep = pl.pallas_call(
    _tc_prep_body,
    out_shape=(jax.ShapeDtypeStruct((N, 1), jnp.float32),
               jax.ShapeDtypeStruct((N, D_HID), jnp.float32)),
)


def _tc_mid_body(accp_ref, g1_ref, dinv_ref, b1_ref, gamma_ref, beta_ref,
                 mean_ref, var_ref, s_ref):
    dinv = dinv_ref[...]
    pre = dinv * (accp_ref[0] + accp_ref[1] + g1_ref[...]) + b1_ref[...]
    scale = gamma_ref[...] * lax.rsqrt(var_ref[...] + 1e-5)
    hb = (pre - mean_ref[...]) * scale + beta_ref[...]
    s_ref[...] = dinv * jnp.maximum(hb, 0.0)


_tc_mid = pl.pallas_call(
    _tc_mid_body,
    out_shape=jax.ShapeDtypeStruct((N, D_HID), jnp.float32),
)


def _tc_out_body(accp_ref, s_ref, dinv_ref, w2_ref, b2_ref, out_ref):
    u = dinv_ref[...] * (accp_ref[0] + accp_ref[1] + s_ref[...])
    out_ref[...] = jnp.dot(u, w2_ref[...],
                           preferred_element_type=jnp.float32) + b2_ref[...]


_tc_out = pl.pallas_call(
    _tc_out_body,
    out_shape=jax.ShapeDtypeStruct((N, D_OUT), jnp.float32),
)


@jax.jit
def kernel(x, edge_index, W1, b1, gamma, beta, running_mean, running_var,
           W2, b2):
    src = edge_index[0].reshape(E // CW, CW)
    dst = edge_index[1].reshape(E // CW, CW)
    z_n = jnp.zeros((N,), jnp.float32)
    z_h = jnp.zeros((N, D_HID), jnp.float32)

    degp = _sc_degree(dst, z_n).reshape(NSC, N, 1)
    dinv, g1 = _tc_prep(degp, x, W1)
    acc1 = _sc_edge_agg(g1, src, dst, z_h).reshape(NSC, N, D_HID)
    s = _tc_mid(acc1, g1, dinv, b1.reshape(1, -1), gamma.reshape(1, -1),
                beta.reshape(1, -1), running_mean.reshape(1, -1),
                running_var.reshape(1, -1))
    acc2 = _sc_edge_agg(s, src, dst, z_h).reshape(NSC, N, D_HID)
    return _tc_out(acc2, s, dinv, W2, b2.reshape(1, -1))
